# baseline (speedup 1.0000x reference)
import functools

import jax
import jax.numpy as jnp
from jax import lax
from jax.experimental import pallas as pl
from jax.experimental.pallas import tpu as pltpu
from jax.experimental.pallas import tpu_sc as plsc

B = 16384
D = 32
NC = 2
NS = 16
NW = NC * NS
BPW = B // NW


def _pmf_body(uidx_hbm, iidx_hbm, uemb_hbm, iemb_hbm, out_hbm,
              uidx_v, iidx_v, urows_v, irows_v, out_v, sem):
    c = lax.axis_index("c")
    s = lax.axis_index("s")
    wid = s * NC + c
    base = wid * BPW

    pltpu.sync_copy(uidx_hbm.at[pl.ds(base, BPW)], uidx_v)
    pltpu.sync_copy(iidx_hbm.at[pl.ds(base, BPW)], iidx_v)

    cu = pltpu.async_copy(uemb_hbm.at[uidx_v], urows_v, sem)
    ci = pltpu.async_copy(iemb_hbm.at[iidx_v], irows_v, sem)
    cu.wait()
    ci.wait()

    lane = lax.iota(jnp.int32, 16)

    def merge(a, b, k):
        perm = lane ^ k
        mask = (lane & k) == 0
        a_s = a.at[perm].get(mode="promise_in_bounds")
        b_s = b.at[perm].get(mode="promise_in_bounds")
        return jnp.where(mask, a, b_s) + jnp.where(mask, a_s, b)

    def group(g, carry):
        base_r = g * 16
        vs = []
        for r in range(16):
            u0 = urows_v[base_r + r, pl.ds(0, 16)]
            i0 = irows_v[base_r + r, pl.ds(0, 16)]
            u1 = urows_v[base_r + r, pl.ds(16, 16)]
            i1 = irows_v[base_r + r, pl.ds(16, 16)]
            vs.append(u0 * i0 + u1 * i1)
        for k in (1, 2, 4, 8):
            vs = [merge(vs[2 * j], vs[2 * j + 1], k)
                  for j in range(len(vs) // 2)]
        out_v[pl.ds(base_r, 16)] = vs[0]
        return carry

    lax.fori_loop(0, BPW // 16, group, 0)

    pltpu.sync_copy(out_v, out_hbm.at[pl.ds(base, BPW)])


@jax.jit
def _pmf(users_index, items_index, user_emb, item_emb):
    mesh = plsc.VectorSubcoreMesh(core_axis_name="c", subcore_axis_name="s")
    f = functools.partial(
        pl.kernel,
        mesh=mesh,
        out_type=jax.ShapeDtypeStruct((B,), jnp.float32),
        compiler_params=pltpu.CompilerParams(use_tc_tiling_on_sc=False),
        scratch_types=[
            pltpu.VMEM((BPW,), jnp.int32),
            pltpu.VMEM((BPW,), jnp.int32),
            pltpu.VMEM((BPW, D), jnp.float32),
            pltpu.VMEM((BPW, D), jnp.float32),
            pltpu.VMEM((BPW,), jnp.float32),
            pltpu.SemaphoreType.DMA,
        ],
    )(_pmf_body)
    return f(users_index, items_index, user_emb, item_emb)


def kernel(users_index, items_index, user_emb, item_emb):
    return _pmf(users_index, items_index, user_emb, item_emb)


# speedup vs baseline: 3.4726x; 3.4726x over previous
import functools

import jax
import jax.numpy as jnp
from jax import lax
from jax.experimental import pallas as pl
from jax.experimental.pallas import tpu as pltpu
from jax.experimental.pallas import tpu_sc as plsc

B = 16384
D = 32
NC = 2
NS = 16
NW = NC * NS
BPW = B // NW
CHUNK = 16
LANES = 128


def _pmf_body(uidx_hbm, iidx_hbm, uemb_hbm, iemb_hbm, out_hbm,
              uidx_v, iidx_v, buf_v, ucol_v, icol_v, out_v, sem):
    c = lax.axis_index("c")
    s = lax.axis_index("s")
    wid = s * NC + c
    base = wid * BPW

    pltpu.sync_copy(uidx_hbm.at[pl.ds(base, BPW)], uidx_v)
    pltpu.sync_copy(iidx_hbm.at[pl.ds(base, BPW)], iidx_v)

    lane = lax.iota(jnp.int32, 16)

    def merge(a, b, k):
        perm = lane ^ k
        mask = (lane & k) == 0
        a_s = a.at[perm].get(mode="promise_in_bounds")
        b_s = b.at[perm].get(mode="promise_in_bounds")
        return jnp.where(mask, a, b_s) + jnp.where(mask, a_s, b)

    def fetch_extract(emb_hbm, idx_vec, col_ref):
        blk = (idx_vec >> 7) << 7
        lo = idx_vec & (LANES - 1)
        copies = []
        for m in range(CHUNK):
            start = pl.multiple_of(blk[m], LANES)
            copies.append(pltpu.async_copy(
                emb_hbm.at[:, pl.ds(start, LANES)],
                buf_v.at[pl.ds(m * D, D)], sem))
        for cp in copies:
            cp.wait()
        for m in range(CHUNK):
            lm = jnp.full((16,), lo[m], jnp.int32)
            g0 = plsc.load_gather(buf_v, [lane + m * D, lm])
            g1 = plsc.load_gather(buf_v, [lane + (m * D + 16), lm])
            col_ref[m, pl.ds(0, 16)] = g0
            col_ref[m, pl.ds(16, 16)] = g1

    def chunk(g, carry):
        cb = g * CHUNK
        uvec = uidx_v[pl.ds(cb, 16)]
        ivec = iidx_v[pl.ds(cb, 16)]
        fetch_extract(uemb_hbm, uvec, ucol_v)
        fetch_extract(iemb_hbm, ivec, icol_v)
        vs = []
        for m in range(CHUNK):
            u0 = ucol_v[m, pl.ds(0, 16)]
            i0 = icol_v[m, pl.ds(0, 16)]
            u1 = ucol_v[m, pl.ds(16, 16)]
            i1 = icol_v[m, pl.ds(16, 16)]
            vs.append(u0 * i0 + u1 * i1)
        for k in (1, 2, 4, 8):
            vs = [merge(vs[2 * j], vs[2 * j + 1], k)
                  for j in range(len(vs) // 2)]
        out_v[pl.ds(cb, 16)] = vs[0]
        return carry

    lax.fori_loop(0, BPW // CHUNK, chunk, 0)

    pltpu.sync_copy(out_v, out_hbm.at[pl.ds(base, BPW)])


@jax.jit
def _pmf(users_index, items_index, user_emb_t, item_emb_t):
    mesh = plsc.VectorSubcoreMesh(core_axis_name="c", subcore_axis_name="s")
    f = functools.partial(
        pl.kernel,
        mesh=mesh,
        out_type=jax.ShapeDtypeStruct((B,), jnp.float32),
        compiler_params=pltpu.CompilerParams(needs_layout_passes=False),
        scratch_types=[
            pltpu.VMEM((BPW,), jnp.int32),
            pltpu.VMEM((BPW,), jnp.int32),
            pltpu.VMEM((CHUNK * D, LANES), jnp.float32),
            pltpu.VMEM((CHUNK, D), jnp.float32),
            pltpu.VMEM((CHUNK, D), jnp.float32),
            pltpu.VMEM((BPW,), jnp.float32),
            pltpu.SemaphoreType.DMA,
        ],
    )(_pmf_body)
    return f(users_index, items_index, user_emb_t, item_emb_t)


def kernel(users_index, items_index, user_emb, item_emb):
    return _pmf(users_index, items_index, user_emb.T, item_emb.T)


# speedup vs baseline: 3.6634x; 1.0550x over previous
import functools

import jax
import jax.numpy as jnp
from jax import lax
from jax.experimental import pallas as pl
from jax.experimental.pallas import tpu as pltpu
from jax.experimental.pallas import tpu_sc as plsc

B = 16384
D = 32
NC = 2
NS = 16
NW = NC * NS
BPW = B // NW
UNIT = 4
LANES = 128
VOCAB = 1000000
MAXBLK = ((VOCAB + LANES - 1) // LANES - 1) * LANES


def _pmf_body(uidx_hbm, iidx_hbm, uemb_hbm, iemb_hbm, out_hbm,
              uidx_v, iidx_v, ub0, ib0, ub1, ib1, out_v, sem_a, sem_b):
    c = lax.axis_index("c")
    s = lax.axis_index("s")
    wid = s * NC + c
    base = wid * BPW

    pltpu.sync_copy(uidx_hbm.at[pl.ds(base, BPW)], uidx_v.at[pl.ds(0, BPW)])
    pltpu.sync_copy(iidx_hbm.at[pl.ds(base, BPW)], iidx_v.at[pl.ds(0, BPW)])

    lane = lax.iota(jnp.int32, 16)

    def merge(a, b, k):
        perm = lane ^ k
        mask = (lane & k) == 0
        a_s = a.at[perm].get(mode="promise_in_bounds")
        b_s = b.at[perm].get(mode="promise_in_bounds")
        return jnp.where(mask, a, b_s) + jnp.where(mask, a_s, b)

    def issue(n, ubuf, ibuf, sem):
        off = n * UNIT
        uv = uidx_v[pl.ds(off, 16)]
        iv = iidx_v[pl.ds(off, 16)]
        ublk = jnp.clip((uv >> 7) << 7, 0, MAXBLK)
        iblk = jnp.clip((iv >> 7) << 7, 0, MAXBLK)
        for m in range(UNIT):
            su = pl.multiple_of(ublk[m], LANES)
            si = pl.multiple_of(iblk[m], LANES)
            pltpu.async_copy(uemb_hbm.at[:, pl.ds(su, LANES)],
                             ubuf.at[pl.ds(m * D, D)], sem)
            pltpu.async_copy(iemb_hbm.at[:, pl.ds(si, LANES)],
                             ibuf.at[pl.ds(m * D, D)], sem)

    def wait_unit(ubuf, ibuf, sem):
        for m in range(UNIT):
            pltpu.make_async_copy(uemb_hbm.at[:, pl.ds(0, LANES)],
                                  ubuf.at[pl.ds(m * D, D)], sem).wait()
            pltpu.make_async_copy(iemb_hbm.at[:, pl.ds(0, LANES)],
                                  ibuf.at[pl.ds(m * D, D)], sem).wait()

    def process(n, ubuf, ibuf):
        off = n * UNIT
        uv = uidx_v[pl.ds(off, 16)]
        iv = iidx_v[pl.ds(off, 16)]
        ulo = uv & (LANES - 1)
        ilo = iv & (LANES - 1)
        ps = []
        for m in range(UNIT):
            ulm = jnp.full((16,), ulo[m], jnp.int32)
            ilm = jnp.full((16,), ilo[m], jnp.int32)
            u0 = plsc.load_gather(ubuf, [lane + m * D, ulm])
            u1 = plsc.load_gather(ubuf, [lane + (m * D + 16), ulm])
            i0 = plsc.load_gather(ibuf, [lane + m * D, ilm])
            i1 = plsc.load_gather(ibuf, [lane + (m * D + 16), ilm])
            ps.append(u0 * i0 + u1 * i1)
        return ps

    issue(0, ub0, ib0, sem_a)

    def body(g, carry):
        n0 = g * 4
        vs = []
        for k in range(4):
            n = n0 + k
            nxt = n + 1
            if k % 2 == 0:
                issue(nxt, ub1, ib1, sem_b)
                wait_unit(ub0, ib0, sem_a)
                vs.extend(process(n, ub0, ib0))
            else:
                issue(nxt, ub0, ib0, sem_a)
                wait_unit(ub1, ib1, sem_b)
                vs.extend(process(n, ub1, ib1))
        for k in (1, 2, 4, 8):
            vs = [merge(vs[2 * j], vs[2 * j + 1], k)
                  for j in range(len(vs) // 2)]
        out_v[pl.ds(n0 * UNIT, 16)] = vs[0]
        return carry

    lax.fori_loop(0, BPW // (4 * UNIT), body, 0)

    wait_unit(ub0, ib0, sem_a)

    pltpu.sync_copy(out_v, out_hbm.at[pl.ds(base, BPW)])


@jax.jit
def _pmf(users_index, items_index, user_emb_t, item_emb_t):
    mesh = plsc.VectorSubcoreMesh(core_axis_name="c", subcore_axis_name="s")
    f = functools.partial(
        pl.kernel,
        mesh=mesh,
        out_type=jax.ShapeDtypeStruct((B,), jnp.float32),
        compiler_params=pltpu.CompilerParams(needs_layout_passes=False),
        scratch_types=[
            pltpu.VMEM((BPW + 16,), jnp.int32),
            pltpu.VMEM((BPW + 16,), jnp.int32),
            pltpu.VMEM((UNIT * D, LANES), jnp.float32),
            pltpu.VMEM((UNIT * D, LANES), jnp.float32),
            pltpu.VMEM((UNIT * D, LANES), jnp.float32),
            pltpu.VMEM((UNIT * D, LANES), jnp.float32),
            pltpu.VMEM((BPW,), jnp.float32),
            pltpu.SemaphoreType.DMA,
            pltpu.SemaphoreType.DMA,
        ],
    )(_pmf_body)
    return f(users_index, items_index, user_emb_t, item_emb_t)


def kernel(users_index, items_index, user_emb, item_emb):
    return _pmf(users_index, items_index, user_emb.T, item_emb.T)
